# baseline (speedup 1.0000x reference)
import functools

import jax
import jax.numpy as jnp
from jax import lax
from jax.experimental import pallas as pl
from jax.experimental.pallas import tpu as pltpu
from jax.experimental.pallas import tpu_sc as plsc

_TEMPERATURE = 0.07
_NC = 2
_NS = 16
_NW = _NC * _NS
_CH = 64
_L = 512


def _sc_reduce_body(row0_base, n_sc_rows, h_ref, y_ref, sums_ref, cnt_ref,
                    buf0, buf1, acc, cbuf, ybuf, sem0, sem1):
    wid = lax.axis_index("s") * _NC + lax.axis_index("c")
    rpw = n_sc_rows // _NW
    base = row0_base + wid * rpw
    nch = rpw // _CH
    zero = jnp.zeros((16,), jnp.float32)
    for j in range(2 * _L // 16):
        acc[pl.ds(16 * j, 16)] = zero

    pltpu.sync_copy(y_ref.at[pl.ds(base, rpw)], ybuf)

    bufs = (buf0, buf1)
    sems = (sem0, sem1)

    def start(ci, b):
        pltpu.async_copy(
            h_ref.at[pl.ds(base + ci * _CH, _CH), :], bufs[b], sems[b])

    def wait(b):
        pltpu.make_async_copy(
            h_ref.at[pl.ds(base, _CH), :], bufs[b], sems[b]).wait()

    def process(ci, b, cntv):
        buf = bufs[b]

        def group_body(g, cntv):
            yv = ybuf[pl.ds(ci * _CH + g * 16, 16)]
            yf = yv.astype(jnp.float32)
            gr = g * 16
            wts = [jnp.full((16,), yf[e], jnp.float32) for e in range(16)]
            for j in range(_L // 16):
                col = 16 * j
                a_all = buf[gr, pl.ds(col, 16)]
                a_one = a_all * wts[0]
                for e in range(1, 16):
                    v = buf[gr + e, pl.ds(col, 16)]
                    a_all = a_all + v
                    a_one = a_one + v * wts[e]
                plsc.addupdate(acc.at[pl.ds(col, 16)], a_all)
                plsc.addupdate(acc.at[pl.ds(_L + col, 16)], a_one)
            return cntv + yv

        return lax.fori_loop(0, _CH // 16, group_body, cntv)

    start(0, 0)
    cntv = jnp.zeros((16,), jnp.int32)

    def outer(ci2, cntv):
        for b in range(2):
            ci = 2 * ci2 + b

            @pl.when(ci + 1 < nch)
            def _():
                start(ci + 1, 1 - b)

            wait(b)
            cntv = process(ci, b, cntv)
        return cntv

    cntv = lax.fori_loop(0, nch // 2, outer, cntv)
    pltpu.sync_copy(acc, sums_ref.at[wid])
    cbuf[...] = cntv.astype(jnp.float32)
    pltpu.sync_copy(cbuf, cnt_ref.at[wid])


def _sc_reduce(h1, y, row0_base, n_sc_rows):
    rpw = n_sc_rows // _NW
    mesh = plsc.VectorSubcoreMesh(
        core_axis_name="c", subcore_axis_name="s",
        num_cores=_NC, num_subcores=_NS)
    f = pl.kernel(
        functools.partial(_sc_reduce_body, row0_base, n_sc_rows),
        out_type=[
            jax.ShapeDtypeStruct((_NW, 2 * _L), jnp.float32),
            jax.ShapeDtypeStruct((_NW, 16), jnp.float32),
        ],
        mesh=mesh,
        cost_estimate=pl.CostEstimate(
            flops=4 * n_sc_rows * _L * 40,
            bytes_accessed=4 * n_sc_rows * _L * 10,
            transcendentals=0),
        scratch_types=[
            pltpu.VMEM((_CH, _L), jnp.float32),
            pltpu.VMEM((_CH, _L), jnp.float32),
            pltpu.VMEM((2 * _L,), jnp.float32),
            pltpu.VMEM((16,), jnp.float32),
            pltpu.VMEM((rpw,), jnp.int32),
            pltpu.SemaphoreType.DMA,
            pltpu.SemaphoreType.DMA,
        ],
    )
    return f(h1, y)


_TC_CHUNK = 2048


def _tc_partial_body(n_steps, y_ref, h_ref, sums_ref, cnt_ref, acc_ref):
    c = pl.program_id(0)

    @pl.when(c == 0)
    def _init():
        acc_ref[...] = jnp.zeros_like(acc_ref)

    hb = h_ref[...]
    yb = y_ref[...]
    ym = yb.reshape(_TC_CHUNK // 128, 128, 1) > 0.5
    sel = jnp.where(ym, hb.reshape(_TC_CHUNK // 128, 128, _L), 0.0)
    h3 = hb.reshape(_TC_CHUNK // 8, 8, _L)
    s3 = sel.reshape(_TC_CHUNK // 8, 8, _L)
    acc_ref[0:8, :] += jnp.sum(h3, axis=0)
    acc_ref[8:16, :] += jnp.sum(s3, axis=0)
    acc_ref[16:17, 0:128] += yb.reshape(_TC_CHUNK // 128, 128).sum(
        axis=0, keepdims=True)

    @pl.when(c == n_steps - 1)
    def _emit():
        sums_ref[...] = acc_ref[0:16, :]
        cnt_ref[...] = acc_ref[16:17, 0:128]


def _tc_partial(h2, yw, n_tc_rows):
    l = h2.shape[1]
    n_steps = n_tc_rows // _TC_CHUNK
    out = pl.pallas_call(
        functools.partial(_tc_partial_body, n_steps),
        grid=(n_steps,),
        in_specs=[
            pl.BlockSpec((_TC_CHUNK // 128, 128), lambda c: (c, 0)),
            pl.BlockSpec((_TC_CHUNK, l), lambda c: (c, 0)),
        ],
        out_specs=[
            pl.BlockSpec((16, l), lambda c: (0, 0)),
            pl.BlockSpec((1, 128), lambda c: (0, 0)),
        ],
        out_shape=[
            jax.ShapeDtypeStruct((16, l), jnp.float32),
            jax.ShapeDtypeStruct((1, 128), jnp.float32),
        ],
        scratch_shapes=[
            pltpu.VMEM((17, l), jnp.float32),
        ],
    )(yw, h2)
    return out


def _loss_from_sums(n_rows, p_sz, p_nsz, s_all, s_y, n1,
                    emb, psz_ref, pnsz_ref):
    n0 = jnp.float32(n_rows) - n1
    m_sz = s_y / jnp.maximum(n1, 1.0)
    m_nsz = (s_all - s_y) / jnp.maximum(n0, 1.0)
    m_sz = m_sz / jnp.maximum(jnp.sqrt(jnp.sum(m_sz * m_sz)), 1e-12)
    m_nsz = m_nsz / jnp.maximum(jnp.sqrt(jnp.sum(m_nsz * m_nsz)), 1e-12)

    z_sz = lax.dot_general(
        m_sz, emb, (((1,), (1,)), ((), ())),
        preferred_element_type=jnp.float32) / _TEMPERATURE
    z_nsz = lax.dot_general(
        m_nsz, emb, (((1,), (1,)), ((), ())),
        preferred_element_type=jnp.float32) / _TEMPERATURE

    sim_sz = jnp.exp(z_sz)
    sim_nsz = jnp.exp(z_nsz)
    lanes = lax.broadcasted_iota(jnp.int32, z_sz.shape, 1)

    mask_sz = jnp.zeros(z_sz.shape, dtype=jnp.bool_)
    zsum_sz = jnp.float32(0.0)
    for i in range(p_sz):
        hit = lanes == psz_ref[i]
        mask_sz = jnp.logical_or(mask_sz, hit)
        zsum_sz += jnp.sum(jnp.where(hit, z_sz, 0.0))
    den_sz = jnp.sum(jnp.where(mask_sz, 0.0, sim_sz))
    loss_sz = jnp.log(den_sz) - zsum_sz / jnp.float32(p_sz)

    mask_nsz = jnp.zeros(z_nsz.shape, dtype=jnp.bool_)
    zsum_nsz = jnp.float32(0.0)
    for i in range(p_nsz):
        hit = lanes == pnsz_ref[i]
        mask_nsz = jnp.logical_or(mask_nsz, hit)
        zsum_nsz += jnp.sum(jnp.where(hit, z_nsz, 0.0))
    den_nsz = jnp.sum(jnp.where(mask_nsz, 0.0, sim_nsz))
    loss_nsz = jnp.log(den_nsz) - zsum_nsz / jnp.float32(p_nsz)

    return loss_sz + loss_nsz


def _epilogue_body(n_rows, p_sz, p_nsz,
                   sc_sums_ref, sc_cnt_ref, tc_sums_ref, tc_cnt_ref,
                   emb_ref, psz_ref, pnsz_ref, out_ref):
    sc = sc_sums_ref[...]
    tc = tc_sums_ref[...]
    s_all = (jnp.sum(sc[:, 0:_L], axis=0, keepdims=True)
             + jnp.sum(tc[0:8, :], axis=0, keepdims=True))
    s1 = (jnp.sum(sc[:, _L:2 * _L], axis=0, keepdims=True)
          + jnp.sum(tc[8:16, :], axis=0, keepdims=True))
    n1 = jnp.sum(sc_cnt_ref[...]) + jnp.sum(tc_cnt_ref[...])
    loss = _loss_from_sums(n_rows, p_sz, p_nsz, s_all, s1, n1,
                           emb_ref[...], psz_ref, pnsz_ref)
    out_ref[...] = jnp.broadcast_to(loss, (1, 1))


_N_SC = 8192


def kernel(hg, all_emb, det_labels, concept_labels, Psz_idx, Pnsz_idx):
    del concept_labels
    b, nsz, t, l = hg.shape
    n = b * nsz * t
    h2 = hg.reshape(n, l)
    y = det_labels.reshape(n)
    yw = det_labels.reshape(n // 128, 128).astype(jnp.float32)

    n_tc = n - _N_SC
    sc_sums, sc_cnts = _sc_reduce(h2, y, n_tc, _N_SC)
    tc_sums, tc_cnts = _tc_partial(h2, yw, n_tc)

    p_sz = int(Psz_idx.shape[0])
    p_nsz = int(Pnsz_idx.shape[0])
    out = pl.pallas_call(
        functools.partial(_epilogue_body, n, p_sz, p_nsz),
        in_specs=[
            pl.BlockSpec((_NW, 2 * _L), lambda: (0, 0)),
            pl.BlockSpec((_NW, 16), lambda: (0, 0)),
            pl.BlockSpec((16, _L), lambda: (0, 0)),
            pl.BlockSpec((1, 128), lambda: (0, 0)),
            pl.BlockSpec((all_emb.shape[0], l), lambda: (0, 0)),
            pl.BlockSpec(memory_space=pltpu.SMEM),
            pl.BlockSpec(memory_space=pltpu.SMEM),
        ],
        out_specs=pl.BlockSpec((1, 1), lambda: (0, 0)),
        out_shape=jax.ShapeDtypeStruct((1, 1), jnp.float32),
    )(sc_sums, sc_cnts, tc_sums, tc_cnts, all_emb, Psz_idx, Pnsz_idx)
    return out[0, 0]


# speedup vs baseline: 1.3264x; 1.3264x over previous
import functools

import jax
import jax.numpy as jnp
from jax.experimental import pallas as pl
from jax.experimental.pallas import tpu as pltpu

_TEMPERATURE = 0.07
_CHUNK = 2048


def _body(n_rows, n_steps, p_sz, p_nsz,
          y_ref, h_ref, emb_ref, psz_ref, pnsz_ref,
          out_ref, acc_ref, cnt_ref):
    c = pl.program_id(0)

    @pl.when(c == 0)
    def _init():
        acc_ref[...] = jnp.zeros_like(acc_ref)
        cnt_ref[0] = jnp.float32(0.0)

    hb = h_ref[...]
    yb = y_ref[...]
    ym = yb.reshape(_CHUNK // 128, 128, 1) > 0.5
    sel = jnp.where(ym, hb.reshape(_CHUNK // 128, 128, 512), 0.0)
    h3 = hb.reshape(_CHUNK // 8, 8, 512)
    s3 = sel.reshape(_CHUNK // 8, 8, 512)
    acc_ref[0:8, :] += jnp.sum(h3, axis=0)
    acc_ref[8:16, :] += jnp.sum(s3, axis=0)
    cnt_ref[0] += jnp.sum(yb)

    @pl.when(c == n_steps - 1)
    def _epilogue():
        s_all = jnp.sum(acc_ref[0:8, :], axis=0, keepdims=True)
        s_y = jnp.sum(acc_ref[8:16, :], axis=0, keepdims=True)
        n1 = cnt_ref[0]
        n0 = jnp.float32(n_rows) - n1
        m_sz = s_y / jnp.maximum(n1, 1.0)
        m_nsz = (s_all - s_y) / jnp.maximum(n0, 1.0)
        m_sz = m_sz / jnp.maximum(jnp.sqrt(jnp.sum(m_sz * m_sz)), 1e-12)
        m_nsz = m_nsz / jnp.maximum(jnp.sqrt(jnp.sum(m_nsz * m_nsz)), 1e-12)

        emb = emb_ref[...]
        z_sz = jax.lax.dot_general(
            m_sz, emb, (((1,), (1,)), ((), ())),
            preferred_element_type=jnp.float32) / _TEMPERATURE
        z_nsz = jax.lax.dot_general(
            m_nsz, emb, (((1,), (1,)), ((), ())),
            preferred_element_type=jnp.float32) / _TEMPERATURE

        sim_sz = jnp.exp(z_sz)
        sim_nsz = jnp.exp(z_nsz)
        lanes = jax.lax.broadcasted_iota(jnp.int32, z_sz.shape, 1)

        mask_sz = jnp.zeros(z_sz.shape, dtype=jnp.bool_)
        zsum_sz = jnp.float32(0.0)
        for i in range(p_sz):
            hit = lanes == psz_ref[i]
            mask_sz = jnp.logical_or(mask_sz, hit)
            zsum_sz += jnp.sum(jnp.where(hit, z_sz, 0.0))
        den_sz = jnp.sum(jnp.where(mask_sz, 0.0, sim_sz))
        loss_sz = jnp.log(den_sz) - zsum_sz / jnp.float32(p_sz)

        mask_nsz = jnp.zeros(z_nsz.shape, dtype=jnp.bool_)
        zsum_nsz = jnp.float32(0.0)
        for i in range(p_nsz):
            hit = lanes == pnsz_ref[i]
            mask_nsz = jnp.logical_or(mask_nsz, hit)
            zsum_nsz += jnp.sum(jnp.where(hit, z_nsz, 0.0))
        den_nsz = jnp.sum(jnp.where(mask_nsz, 0.0, sim_nsz))
        loss_nsz = jnp.log(den_nsz) - zsum_nsz / jnp.float32(p_nsz)

        out_ref[...] = jnp.broadcast_to(loss_sz + loss_nsz, (1, 1))


def kernel(hg, all_emb, det_labels, concept_labels, Psz_idx, Pnsz_idx):
    del concept_labels
    b, nsz, t, l = hg.shape
    n = b * nsz * t
    h2 = hg.reshape(n, l)
    yf = det_labels.reshape(n // 128, 128).astype(jnp.float32)

    n_steps = n // _CHUNK
    p_sz = int(Psz_idx.shape[0])
    p_nsz = int(Pnsz_idx.shape[0])

    body = functools.partial(_body, n, n_steps, p_sz, p_nsz)

    out = pl.pallas_call(
        body,
        grid=(n_steps,),
        in_specs=[
            pl.BlockSpec((_CHUNK // 128, 128), lambda c: (c, 0)),
            pl.BlockSpec((_CHUNK, l), lambda c: (c, 0)),
            pl.BlockSpec((all_emb.shape[0], l), lambda c: (0, 0)),
            pl.BlockSpec(memory_space=pltpu.SMEM),
            pl.BlockSpec(memory_space=pltpu.SMEM),
        ],
        out_specs=pl.BlockSpec((1, 1), lambda c: (0, 0)),
        out_shape=jax.ShapeDtypeStruct((1, 1), jnp.float32),
        scratch_shapes=[
            pltpu.VMEM((16, l), jnp.float32),
            pltpu.SMEM((1,), jnp.float32),
        ],
    )(yf, h2, all_emb, Psz_idx, Pnsz_idx)
    return out[0, 0]
